# baseline (speedup 1.0000x reference)
import functools
import jax
import jax.numpy as jnp
from jax import lax
from jax.experimental import pallas as pl
from jax.experimental.pallas import tpu as pltpu
from jax.experimental.pallas import tpu_sc as plsc

_T, _D, _H, _E = 2048, 1024, 2048, 8
_K = 2
_S = _T * _K
_B = 128
_NB = _S // _B + _E
_NR = _NB * _B
_HB = 512
_NH = _H // _HB
_NW = 32
_RPW = _NR // _NW
_GC = 32

_mesh = plsc.VectorSubcoreMesh(core_axis_name="c", subcore_axis_name="s")


def _gate_body(x_ref, wg_ref, eidx_ref, wf_ref):
    logits = lax.dot_general(wg_ref[...], x_ref[...], (((1,), (1,)), ((), ())),
                             preferred_element_type=jnp.float32)
    ei = lax.broadcasted_iota(jnp.int32, logits.shape, 0)
    m1 = jnp.max(logits, axis=0, keepdims=True)
    a1 = jnp.min(jnp.where(logits == m1, ei, _E), axis=0, keepdims=True)
    l2 = jnp.where(ei == a1, -jnp.inf, logits)
    m2 = jnp.max(l2, axis=0, keepdims=True)
    a2 = jnp.min(jnp.where(l2 == m2, ei, _E), axis=0, keepdims=True)
    w1 = 1.0 / (1.0 + jnp.exp(m2 - m1))
    eidx_ref[...] = jnp.concatenate([a1, a2], axis=0)
    wf_ref[...] = jnp.concatenate([w1, 1.0 - w1], axis=0)


def _route_body(eidx_hbm, ppos_hbm, rtok_hbm, bexp_hbm,
                idx_v, pos_v, rtok_v, hist_v, run_v, bs_v, bexp_v):
    wid = lax.axis_index("s") * 2 + lax.axis_index("c")

    @pl.when(wid == 0)
    def _():
        pltpu.sync_copy(eidx_hbm, idx_v)
        lanes = lax.iota(jnp.int32, 16)
        hist_v[...] = jnp.zeros((16,), jnp.int32)

        def slot_vec(j):
            return plsc.load_gather(idx_v, [j * 16 + lanes])

        def hist_step(j, carry):
            v = slot_vec(j)
            cnts = jnp.zeros((16,), jnp.int32)
            for e in range(_E):
                pc = jnp.sum(jnp.where(v == e, 1, 0))
                cnts = jnp.where(lanes == e, pc, cnts)
            hist_v[...] = hist_v[...] + cnts
            return carry

        lax.fori_loop(0, _S // 16, hist_step, 0)

        counts = hist_v[...]
        padded = ((counts + (_B - 1)) >> 7) << 7
        cs = plsc.cumsum(padded)
        gs = cs - padded
        run_v[...] = gs
        bs_v[...] = gs >> 7

        def zero_step(j, carry):
            plsc.store_scatter(rtok_v, [j * 16 + lanes],
                               jnp.zeros((16,), jnp.int32))
            return carry

        lax.fori_loop(0, _NR // 16, zero_step, 0)

        def pos_step(j, carry):
            v = slot_vec(j)
            rank = jnp.zeros((16,), jnp.int32)
            cnts = jnp.zeros((16,), jnp.int32)
            for e in range(_E):
                m = v == e
                c = plsc.cumsum(jnp.where(m, 1, 0))
                rank = jnp.where(m, c - 1, rank)
                pc = jnp.max(c)
                cnts = jnp.where(lanes == e, pc, cnts)
            base = plsc.load_gather(run_v, [v])
            pos = base + rank
            tok = (j * 16 + lanes) & (_T - 1)
            plsc.store_scatter(pos_v, [j * 16 + lanes], pos)
            plsc.store_scatter(rtok_v, [pos], tok)
            run_v[...] = run_v[...] + cnts
            return carry

        lax.fori_loop(0, _S // 16, pos_step, 0)

        for jb in range(8):
            bid = lax.iota(jnp.int32, 16) + jb * 16
            be = jnp.zeros((16,), jnp.int32)
            for e in range(1, _E):
                bs_e = plsc.load_gather(bs_v, [jnp.full((16,), e, jnp.int32)])
                be = be + jnp.where(bid >= bs_e, 1, 0)
            plsc.store_scatter(bexp_v, [jb * 16 + lanes], be)

        pltpu.sync_copy(pos_v, ppos_hbm)
        pltpu.sync_copy(rtok_v, rtok_hbm)
        pltpu.sync_copy(bexp_v, bexp_hbm)


_route_call = functools.partial(
    pl.kernel,
    out_type=(
        jax.ShapeDtypeStruct((_S,), jnp.int32),
        jax.ShapeDtypeStruct((_NR,), jnp.int32),
        jax.ShapeDtypeStruct((128,), jnp.int32),
    ),
    mesh=_mesh,
    compiler_params=pltpu.CompilerParams(needs_layout_passes=False),
    scratch_types=[
        pltpu.VMEM((_S,), jnp.int32),
        pltpu.VMEM((_S,), jnp.int32),
        pltpu.VMEM((_NR,), jnp.int32),
        pltpu.VMEM((16,), jnp.int32),
        pltpu.VMEM((16,), jnp.int32),
        pltpu.VMEM((16,), jnp.int32),
        pltpu.VMEM((128,), jnp.int32),
    ],
)(_route_body)


def _gather_body(x_hbm, rtok_hbm, xs_hbm, idx0, idx1, rows0, rows1,
                 sem0, sem1):
    wid = lax.axis_index("s") * 2 + lax.axis_index("c")
    base = wid * _RPW
    bufs = [(idx0, rows0, sem0), (idx1, rows1, sem1)]
    prev = None
    for c in range(_RPW // _GC):
        iv, r, s = bufs[c % 2]
        start = base + c * _GC
        pltpu.sync_copy(rtok_hbm.at[pl.ds(start, _GC)], iv)
        cp = pltpu.async_copy(x_hbm.at[iv], r, s)
        if prev is not None:
            pcp, pr, pc0 = prev
            pcp.wait()
            pltpu.sync_copy(pr, xs_hbm.at[pl.ds(base + pc0 * _GC, _GC), :])
        prev = (cp, r, c)
    pcp, pr, pc0 = prev
    pcp.wait()
    pltpu.sync_copy(pr, xs_hbm.at[pl.ds(base + pc0 * _GC, _GC), :])


_gather_call = functools.partial(
    pl.kernel,
    out_type=jax.ShapeDtypeStruct((_NR, _D), jnp.float32),
    mesh=_mesh,
    compiler_params=pltpu.CompilerParams(needs_layout_passes=False),
    scratch_types=[
        pltpu.VMEM((_GC,), jnp.int32),
        pltpu.VMEM((_GC,), jnp.int32),
        pltpu.VMEM((_GC, _D), jnp.float32),
        pltpu.VMEM((_GC, _D), jnp.float32),
        pltpu.SemaphoreType.DMA,
        pltpu.SemaphoreType.DMA,
    ],
)(_gather_body)


def _ffn_body(bexp_ref, xs_ref, w1_ref, w3_ref, w2_ref, y_ref,
              w1b, w3b, w2b):
    h = pl.program_id(0)
    b = pl.program_id(1)
    bprev = jnp.maximum(b - 1, 0)
    fresh = jnp.logical_or(
        b == 0,
        bexp_ref[bprev] != bexp_ref[b])

    @pl.when(fresh)
    def _():
        w1b[...] = w1_ref[0].astype(jnp.bfloat16)
        w3b[...] = w3_ref[0].astype(jnp.bfloat16)
        w2b[...] = w2_ref[0].astype(jnp.bfloat16)

    xb = xs_ref[...].astype(jnp.bfloat16)
    a = lax.dot_general(xb, w1b[...], (((1,), (1,)), ((), ())),
                        preferred_element_type=jnp.float32)
    g = lax.dot_general(xb, w3b[...], (((1,), (1,)), ((), ())),
                        preferred_element_type=jnp.float32)
    hh = (a / (1.0 + jnp.exp(-a))) * g
    y = lax.dot_general(hh.astype(jnp.bfloat16), w2b[...],
                        (((1,), (1,)), ((), ())),
                        preferred_element_type=jnp.float32)
    sl = pl.ds(b * _B, _B)

    @pl.when(h == 0)
    def _():
        y_ref[sl, :] = y

    @pl.when(h > 0)
    def _():
        y_ref[sl, :] = y_ref[sl, :] + y


def _combine_body(y_hbm, ppos_hbm, wf_hbm, out_hbm,
                  idx_v, w_v, rows_v, out_v, sem):
    wid = lax.axis_index("s") * 2 + lax.axis_index("c")
    lanes = lax.iota(jnp.int32, 16)
    for c in range(4):
        tb = wid * 64 + c * 16
        pltpu.sync_copy(ppos_hbm.at[pl.ds(tb, 16)], idx_v.at[pl.ds(0, 16)])
        pltpu.sync_copy(ppos_hbm.at[pl.ds(_T + tb, 16)],
                        idx_v.at[pl.ds(16, 16)])
        pltpu.sync_copy(wf_hbm.at[pl.ds(tb, 16)], w_v.at[pl.ds(0, 16)])
        pltpu.sync_copy(wf_hbm.at[pl.ds(_T + tb, 16)],
                        w_v.at[pl.ds(16, 16)])
        pltpu.async_copy(y_hbm.at[idx_v], rows_v, sem).wait()

        def tok_step(ti, carry):
            w0 = plsc.load_gather(w_v, [jnp.full((16,), 0, jnp.int32) + ti])
            w1 = plsc.load_gather(w_v, [jnp.full((16,), 16, jnp.int32) + ti])
            for cc in range(_D // 16):
                sl = pl.ds(cc * 16, 16)
                out_v[ti, sl] = w0 * rows_v[ti, sl] + w1 * rows_v[ti + 16, sl]
            return carry

        lax.fori_loop(0, 16, tok_step, 0)
        pltpu.sync_copy(out_v, out_hbm.at[pl.ds(tb, 16), :])


_combine_call = functools.partial(
    pl.kernel,
    out_type=jax.ShapeDtypeStruct((_T, _D), jnp.float32),
    mesh=_mesh,
    compiler_params=pltpu.CompilerParams(needs_layout_passes=False),
    scratch_types=[
        pltpu.VMEM((2 * 16,), jnp.int32),
        pltpu.VMEM((2 * 16,), jnp.float32),
        pltpu.VMEM((2 * 16, _D), jnp.float32),
        pltpu.VMEM((16, _D), jnp.float32),
        pltpu.SemaphoreType.DMA,
    ],
)(_combine_body)


def kernel(x, Wg, W1, W2, W3):
    eidx, wf = pl.pallas_call(
        _gate_body,
        out_shape=(jax.ShapeDtypeStruct((_K, _T), jnp.int32),
                   jax.ShapeDtypeStruct((_K, _T), jnp.float32)),
    )(x, Wg)

    ppos, rtok, bexp = _route_call(eidx.reshape(_S))

    xs = _gather_call(x, rtok)

    grid_spec = pltpu.PrefetchScalarGridSpec(
        num_scalar_prefetch=1,
        grid=(_NH, _NB),
        in_specs=[
            pl.BlockSpec((_B, _D), lambda h, b, sref: (b, 0)),
            pl.BlockSpec((1, _HB, _D),
                         lambda h, b, sref: (sref[b], h, 0)),
            pl.BlockSpec((1, _HB, _D),
                         lambda h, b, sref: (sref[b], h, 0)),
            pl.BlockSpec((1, _D, _HB),
                         lambda h, b, sref: (sref[b], 0, h)),
        ],
        out_specs=pl.BlockSpec((_NR, _D), lambda h, b, sref: (0, 0)),
        scratch_shapes=[
            pltpu.VMEM((_HB, _D), jnp.bfloat16),
            pltpu.VMEM((_HB, _D), jnp.bfloat16),
            pltpu.VMEM((_D, _HB), jnp.bfloat16),
        ],
    )
    y = pl.pallas_call(
        _ffn_body,
        grid_spec=grid_spec,
        out_shape=jax.ShapeDtypeStruct((_NR, _D), jnp.float32),
        compiler_params=pltpu.CompilerParams(
            dimension_semantics=("arbitrary", "arbitrary")),
    )(bexp, xs, W1, W3, W2)

    out = _combine_call(y, ppos, wf.reshape(_S))
    return out


# speedup vs baseline: 1.2457x; 1.2457x over previous
import functools
import jax
import jax.numpy as jnp
from jax import lax
from jax.experimental import pallas as pl
from jax.experimental.pallas import tpu as pltpu
from jax.experimental.pallas import tpu_sc as plsc

_T, _D, _H, _E = 2048, 1024, 2048, 8
_K = 2
_S = _T * _K
_B = 256
_NB = _S // _B + _E
_NR = _NB * _B
_HB = 1024
_NH = _H // _HB
_NW = 32
_RPW = _NR // _NW
_GC = 32

_mesh = plsc.VectorSubcoreMesh(core_axis_name="c", subcore_axis_name="s")


def _gate_body(x_ref, wg_ref, eidx_ref, wf_ref):
    logits = lax.dot_general(wg_ref[...], x_ref[...], (((1,), (1,)), ((), ())),
                             preferred_element_type=jnp.float32)
    ei = lax.broadcasted_iota(jnp.int32, logits.shape, 0)
    m1 = jnp.max(logits, axis=0, keepdims=True)
    a1 = jnp.min(jnp.where(logits == m1, ei, _E), axis=0, keepdims=True)
    l2 = jnp.where(ei == a1, -jnp.inf, logits)
    m2 = jnp.max(l2, axis=0, keepdims=True)
    a2 = jnp.min(jnp.where(l2 == m2, ei, _E), axis=0, keepdims=True)
    w1 = 1.0 / (1.0 + jnp.exp(m2 - m1))
    eidx_ref[...] = jnp.concatenate([a1, a2], axis=0)
    wf_ref[...] = jnp.concatenate([w1, 1.0 - w1], axis=0)


def _route_body(eidx_hbm, ppos_hbm, rtok_hbm, bexp_hbm,
                idx_v, pos_v, rtok_v, hist_v, run_v, bs_v, bexp_v):
    wid = lax.axis_index("s") * 2 + lax.axis_index("c")

    @pl.when(wid == 0)
    def _():
        pltpu.sync_copy(eidx_hbm, idx_v)
        lanes = lax.iota(jnp.int32, 16)
        hist_v[...] = jnp.zeros((16,), jnp.int32)

        def slot_vec(j):
            return plsc.load_gather(idx_v, [j * 16 + lanes])

        def hist_step(j, carry):
            v = slot_vec(j)
            cnts = jnp.zeros((16,), jnp.int32)
            for e in range(_E):
                pc = jnp.sum(jnp.where(v == e, 1, 0))
                cnts = jnp.where(lanes == e, pc, cnts)
            hist_v[...] = hist_v[...] + cnts
            return carry

        lax.fori_loop(0, _S // 16, hist_step, 0)

        counts = hist_v[...]
        padded = ((counts + (_B - 1)) >> 8) << 8
        cs = plsc.cumsum(padded)
        gs = cs - padded
        run_v[...] = gs
        bs_v[...] = gs >> 8

        def zero_step(j, carry):
            plsc.store_scatter(rtok_v, [j * 16 + lanes],
                               jnp.zeros((16,), jnp.int32))
            return carry

        lax.fori_loop(0, _NR // 16, zero_step, 0)

        def pos_step(j, carry):
            v = slot_vec(j)
            rank = jnp.zeros((16,), jnp.int32)
            cnts = jnp.zeros((16,), jnp.int32)
            for e in range(_E):
                m = v == e
                c = plsc.cumsum(jnp.where(m, 1, 0))
                rank = jnp.where(m, c - 1, rank)
                pc = jnp.max(c)
                cnts = jnp.where(lanes == e, pc, cnts)
            base = plsc.load_gather(run_v, [v])
            pos = base + rank
            tok = (j * 16 + lanes) & (_T - 1)
            plsc.store_scatter(pos_v, [j * 16 + lanes], pos)
            plsc.store_scatter(rtok_v, [pos], tok)
            run_v[...] = run_v[...] + cnts
            return carry

        lax.fori_loop(0, _S // 16, pos_step, 0)

        for jb in range(8):
            bid = lax.iota(jnp.int32, 16) + jb * 16
            be = jnp.zeros((16,), jnp.int32)
            for e in range(1, _E):
                bs_e = plsc.load_gather(bs_v, [jnp.full((16,), e, jnp.int32)])
                be = be + jnp.where(bid >= bs_e, 1, 0)
            plsc.store_scatter(bexp_v, [jb * 16 + lanes], be)

        pltpu.sync_copy(pos_v, ppos_hbm)
        pltpu.sync_copy(rtok_v, rtok_hbm)
        pltpu.sync_copy(bexp_v, bexp_hbm)


_route_call = functools.partial(
    pl.kernel,
    out_type=(
        jax.ShapeDtypeStruct((_S,), jnp.int32),
        jax.ShapeDtypeStruct((_NR,), jnp.int32),
        jax.ShapeDtypeStruct((128,), jnp.int32),
    ),
    mesh=_mesh,
    compiler_params=pltpu.CompilerParams(needs_layout_passes=False),
    scratch_types=[
        pltpu.VMEM((_S,), jnp.int32),
        pltpu.VMEM((_S,), jnp.int32),
        pltpu.VMEM((_NR,), jnp.int32),
        pltpu.VMEM((16,), jnp.int32),
        pltpu.VMEM((16,), jnp.int32),
        pltpu.VMEM((16,), jnp.int32),
        pltpu.VMEM((128,), jnp.int32),
    ],
)(_route_body)


def _gather_body(x_hbm, rtok_hbm, xs_hbm, idx0, idx1, rows0, rows1,
                 gs0, gs1, ws0, ws1):
    wid = lax.axis_index("s") * 2 + lax.axis_index("c")
    base = wid * _RPW
    idxs = [idx0, idx1]
    rows = [rows0, rows1]
    gsem = [gs0, gs1]
    wsem = [ws0, ws1]
    nch = _RPW // _GC
    gcp = [None] * nch
    wcp = [None] * nch

    def flush(c):
        b = c % 2
        gcp[c].wait()
        wcp[c] = pltpu.async_copy(
            rows[b], xs_hbm.at[pl.ds(base + c * _GC, _GC), :], wsem[b])

    for c in range(nch):
        b = c % 2
        if c >= 2:
            wcp[c - 2].wait()
        pltpu.sync_copy(rtok_hbm.at[pl.ds(base + c * _GC, _GC)], idxs[b])
        gcp[c] = pltpu.async_copy(x_hbm.at[idxs[b]], rows[b], gsem[b])
        if c >= 1:
            flush(c - 1)
    flush(nch - 1)
    wcp[nch - 2].wait()
    wcp[nch - 1].wait()


_gather_call = functools.partial(
    pl.kernel,
    out_type=jax.ShapeDtypeStruct((_NR, _D), jnp.float32),
    mesh=_mesh,
    compiler_params=pltpu.CompilerParams(needs_layout_passes=False),
    scratch_types=[
        pltpu.VMEM((_GC,), jnp.int32),
        pltpu.VMEM((_GC,), jnp.int32),
        pltpu.VMEM((_GC, _D), jnp.float32),
        pltpu.VMEM((_GC, _D), jnp.float32),
        pltpu.SemaphoreType.DMA,
        pltpu.SemaphoreType.DMA,
        pltpu.SemaphoreType.DMA,
        pltpu.SemaphoreType.DMA,
    ],
)(_gather_body)


def _ffn_body(bexp_ref, xs_ref, w1_ref, w3_ref, w2_ref, y_ref,
              w1b, w3b, w2b):
    h = pl.program_id(0)
    b = pl.program_id(1)
    bprev = jnp.maximum(b - 1, 0)
    fresh = jnp.logical_or(
        b == 0,
        bexp_ref[bprev] != bexp_ref[b])

    @pl.when(fresh)
    def _():
        w1b[...] = w1_ref[0].astype(jnp.bfloat16)
        w3b[...] = w3_ref[0].astype(jnp.bfloat16)
        w2b[...] = w2_ref[0].astype(jnp.bfloat16)

    xb = xs_ref[...].astype(jnp.bfloat16)
    a = lax.dot_general(xb, w1b[...], (((1,), (1,)), ((), ())),
                        preferred_element_type=jnp.float32)
    g = lax.dot_general(xb, w3b[...], (((1,), (1,)), ((), ())),
                        preferred_element_type=jnp.float32)
    hh = (a / (1.0 + jnp.exp(-a))) * g
    y = lax.dot_general(hh.astype(jnp.bfloat16), w2b[...],
                        (((1,), (1,)), ((), ())),
                        preferred_element_type=jnp.float32)
    sl = pl.ds(b * _B, _B)

    @pl.when(h == 0)
    def _():
        y_ref[sl, :] = y

    @pl.when(h > 0)
    def _():
        y_ref[sl, :] = y_ref[sl, :] + y


def _combine_body(y_hbm, ppos_hbm, wf_hbm, out_hbm,
                  idx0, idx1, wv0, wv1, rows0, rows1, o0, o1,
                  gs0, gs1, ws0, ws1):
    wid = lax.axis_index("s") * 2 + lax.axis_index("c")
    idxs = [idx0, idx1]
    wvs = [wv0, wv1]
    rows = [rows0, rows1]
    outs = [o0, o1]
    gsem = [gs0, gs1]
    wsem = [ws0, ws1]
    gcp = [None] * 4
    wcp = [None] * 4

    def flush(c):
        b = c % 2
        gcp[c].wait()
        w_v = wvs[b]
        rows_v = rows[b]
        out_v = outs[b]

        def tok_step(ti, carry):
            w0 = plsc.load_gather(w_v, [jnp.full((16,), 0, jnp.int32) + ti])
            w1 = plsc.load_gather(w_v, [jnp.full((16,), 16, jnp.int32) + ti])
            for cc in range(_D // 16):
                sl = pl.ds(cc * 16, 16)
                out_v[ti, sl] = w0 * rows_v[ti, sl] + w1 * rows_v[ti + 16, sl]
            return carry

        lax.fori_loop(0, 16, tok_step, 0)
        wcp[c] = pltpu.async_copy(
            out_v, out_hbm.at[pl.ds(wid * 64 + c * 16, 16), :], wsem[b])

    for c in range(4):
        b = c % 2
        if c >= 2:
            wcp[c - 2].wait()
        tb = wid * 64 + c * 16
        pltpu.sync_copy(ppos_hbm.at[pl.ds(tb, 16)], idxs[b].at[pl.ds(0, 16)])
        pltpu.sync_copy(ppos_hbm.at[pl.ds(_T + tb, 16)],
                        idxs[b].at[pl.ds(16, 16)])
        pltpu.sync_copy(wf_hbm.at[pl.ds(tb, 16)], wvs[b].at[pl.ds(0, 16)])
        pltpu.sync_copy(wf_hbm.at[pl.ds(_T + tb, 16)],
                        wvs[b].at[pl.ds(16, 16)])
        gcp[c] = pltpu.async_copy(y_hbm.at[idxs[b]], rows[b], gsem[b])
        if c >= 1:
            flush(c - 1)
    flush(3)
    wcp[2].wait()
    wcp[3].wait()


_combine_call = functools.partial(
    pl.kernel,
    out_type=jax.ShapeDtypeStruct((_T, _D), jnp.float32),
    mesh=_mesh,
    compiler_params=pltpu.CompilerParams(needs_layout_passes=False),
    scratch_types=[
        pltpu.VMEM((2 * 16,), jnp.int32),
        pltpu.VMEM((2 * 16,), jnp.int32),
        pltpu.VMEM((2 * 16,), jnp.float32),
        pltpu.VMEM((2 * 16,), jnp.float32),
        pltpu.VMEM((2 * 16, _D), jnp.float32),
        pltpu.VMEM((2 * 16, _D), jnp.float32),
        pltpu.VMEM((16, _D), jnp.float32),
        pltpu.VMEM((16, _D), jnp.float32),
        pltpu.SemaphoreType.DMA,
        pltpu.SemaphoreType.DMA,
        pltpu.SemaphoreType.DMA,
        pltpu.SemaphoreType.DMA,
    ],
)(_combine_body)


def kernel(x, Wg, W1, W2, W3):
    eidx, wf = pl.pallas_call(
        _gate_body,
        out_shape=(jax.ShapeDtypeStruct((_K, _T), jnp.int32),
                   jax.ShapeDtypeStruct((_K, _T), jnp.float32)),
    )(x, Wg)

    ppos, rtok, bexp = _route_call(eidx.reshape(_S))

    xs = _gather_call(x, rtok)

    grid_spec = pltpu.PrefetchScalarGridSpec(
        num_scalar_prefetch=1,
        grid=(_NH, _NB),
        in_specs=[
            pl.BlockSpec((_B, _D), lambda h, b, sref: (b, 0)),
            pl.BlockSpec((1, _HB, _D),
                         lambda h, b, sref: (sref[b], h, 0)),
            pl.BlockSpec((1, _HB, _D),
                         lambda h, b, sref: (sref[b], h, 0)),
            pl.BlockSpec((1, _D, _HB),
                         lambda h, b, sref: (sref[b], 0, h)),
        ],
        out_specs=pl.BlockSpec((_NR, _D), lambda h, b, sref: (0, 0)),
        scratch_shapes=[
            pltpu.VMEM((_HB, _D), jnp.bfloat16),
            pltpu.VMEM((_HB, _D), jnp.bfloat16),
            pltpu.VMEM((_D, _HB), jnp.bfloat16),
        ],
    )
    y = pl.pallas_call(
        _ffn_body,
        grid_spec=grid_spec,
        out_shape=jax.ShapeDtypeStruct((_NR, _D), jnp.float32),
        compiler_params=pltpu.CompilerParams(
            dimension_semantics=("arbitrary", "arbitrary")),
    )(bexp, xs, W1, W3, W2)

    out = _combine_call(y, ppos, wf.reshape(_S))
    return out


# speedup vs baseline: 1.2911x; 1.0364x over previous
import functools
import jax
import jax.numpy as jnp
from jax import lax
from jax.experimental import pallas as pl
from jax.experimental.pallas import tpu as pltpu
from jax.experimental.pallas import tpu_sc as plsc

_T, _D, _H, _E = 2048, 1024, 2048, 8
_K = 2
_S = _T * _K
_B = 256
_NB = _S // _B + _E
_NR = _NB * _B
_HB = 1024
_NH = _H // _HB
_NW = 32
_RPW = _NR // _NW
_GC = 32

_mesh = plsc.VectorSubcoreMesh(core_axis_name="c", subcore_axis_name="s")


def _gate_body(x_ref, wg_ref, eidx_ref, wf_ref):
    logits = lax.dot_general(wg_ref[...], x_ref[...], (((1,), (1,)), ((), ())),
                             preferred_element_type=jnp.float32)
    ei = lax.broadcasted_iota(jnp.int32, logits.shape, 0)
    m1 = jnp.max(logits, axis=0, keepdims=True)
    a1 = jnp.min(jnp.where(logits == m1, ei, _E), axis=0, keepdims=True)
    l2 = jnp.where(ei == a1, -jnp.inf, logits)
    m2 = jnp.max(l2, axis=0, keepdims=True)
    a2 = jnp.min(jnp.where(l2 == m2, ei, _E), axis=0, keepdims=True)
    w1 = 1.0 / (1.0 + jnp.exp(m2 - m1))
    eidx_ref[...] = jnp.concatenate([a1, a2], axis=0)
    wf_ref[...] = jnp.concatenate([w1, 1.0 - w1], axis=0)


def _route_body(eidx_hbm, ppos_hbm, rtok_hbm, bexp_hbm,
                idx_v, pos_v, rtok_v, hist_v, run_v, bs_v, bexp_v):
    wid = lax.axis_index("s") * 2 + lax.axis_index("c")

    @pl.when(wid == 0)
    def _():
        pltpu.sync_copy(eidx_hbm, idx_v)
        lanes = lax.iota(jnp.int32, 16)
        hist_v[...] = jnp.zeros((16,), jnp.int32)

        def slot_vec(j):
            return plsc.load_gather(idx_v, [j * 16 + lanes])

        def hist_step(j, carry):
            v = slot_vec(j)
            cnts = jnp.zeros((16,), jnp.int32)
            for e in range(_E):
                pc = jnp.sum(jnp.where(v == e, 1, 0))
                cnts = jnp.where(lanes == e, pc, cnts)
            hist_v[...] = hist_v[...] + cnts
            return carry

        lax.fori_loop(0, _S // 16, hist_step, 0)

        counts = hist_v[...]
        padded = ((counts + (_B - 1)) >> 8) << 8
        cs = plsc.cumsum(padded)
        gs = cs - padded
        run_v[...] = gs
        nact = jnp.max(cs) >> 8
        bs_v[...] = gs >> 8

        def zero_step(j, carry):
            plsc.store_scatter(rtok_v, [j * 16 + lanes],
                               jnp.zeros((16,), jnp.int32))
            return carry

        lax.fori_loop(0, _NR // 16, zero_step, 0)

        def pos_step(j, carry):
            v = slot_vec(j)
            rank = jnp.zeros((16,), jnp.int32)
            cnts = jnp.zeros((16,), jnp.int32)
            for e in range(_E):
                m = v == e
                c = plsc.cumsum(jnp.where(m, 1, 0))
                rank = jnp.where(m, c - 1, rank)
                pc = jnp.max(c)
                cnts = jnp.where(lanes == e, pc, cnts)
            base = plsc.load_gather(run_v, [v])
            pos = base + rank
            tok = (j * 16 + lanes) & (_T - 1)
            plsc.store_scatter(pos_v, [j * 16 + lanes], pos)
            plsc.store_scatter(rtok_v, [pos], tok)
            run_v[...] = run_v[...] + cnts
            return carry

        lax.fori_loop(0, _S // 16, pos_step, 0)

        for jb in range(8):
            bid = lax.iota(jnp.int32, 16) + jb * 16
            be = jnp.zeros((16,), jnp.int32)
            for e in range(1, _E):
                bs_e = plsc.load_gather(bs_v, [jnp.full((16,), e, jnp.int32)])
                be = be + jnp.where(bid >= bs_e, 1, 0)
            plsc.store_scatter(bexp_v, [jb * 16 + lanes], be)
        plsc.store_scatter(bexp_v, [64 + lanes],
                           jnp.zeros((16,), jnp.int32) + nact)

        pltpu.sync_copy(pos_v, ppos_hbm)
        pltpu.sync_copy(rtok_v, rtok_hbm)
        pltpu.sync_copy(bexp_v, bexp_hbm)


_route_call = functools.partial(
    pl.kernel,
    out_type=(
        jax.ShapeDtypeStruct((_S,), jnp.int32),
        jax.ShapeDtypeStruct((_NR,), jnp.int32),
        jax.ShapeDtypeStruct((128,), jnp.int32),
    ),
    mesh=_mesh,
    compiler_params=pltpu.CompilerParams(needs_layout_passes=False),
    scratch_types=[
        pltpu.VMEM((_S,), jnp.int32),
        pltpu.VMEM((_S,), jnp.int32),
        pltpu.VMEM((_NR,), jnp.int32),
        pltpu.VMEM((16,), jnp.int32),
        pltpu.VMEM((16,), jnp.int32),
        pltpu.VMEM((16,), jnp.int32),
        pltpu.VMEM((128,), jnp.int32),
    ],
)(_route_body)


def _gather_body(x_hbm, rtok_hbm, xs_hbm, idx0, idx1, idx2,
                 rows0, rows1, rows2, is0, is1, is2, gs0, gs1, gs2,
                 ws0, ws1, ws2):
    wid = lax.axis_index("s") * 2 + lax.axis_index("c")
    base = wid * _RPW
    idxs = [idx0, idx1, idx2]
    rows = [rows0, rows1, rows2]
    isem = [is0, is1, is2]
    gsem = [gs0, gs1, gs2]
    wsem = [ws0, ws1, ws2]
    nch = _RPW // _GC
    icp = [None] * nch
    gcp = [None] * nch
    wcp = [None] * nch
    for i in range(nch + 2):
        if i < nch:
            b = i % 3
            if i >= 3:
                wcp[i - 3].wait()
            icp[i] = pltpu.async_copy(
                rtok_hbm.at[pl.ds(base + i * _GC, _GC)], idxs[b], isem[b])
        j = i - 1
        if 0 <= j < nch:
            icp[j].wait()
            gcp[j] = pltpu.async_copy(
                x_hbm.at[idxs[j % 3]], rows[j % 3], gsem[j % 3])
        k = i - 2
        if 0 <= k < nch:
            gcp[k].wait()
            wcp[k] = pltpu.async_copy(
                rows[k % 3], xs_hbm.at[pl.ds(base + k * _GC, _GC), :],
                wsem[k % 3])
    wcp[nch - 3].wait()
    wcp[nch - 2].wait()
    wcp[nch - 1].wait()


_gather_call = functools.partial(
    pl.kernel,
    out_type=jax.ShapeDtypeStruct((_NR, _D), jnp.float32),
    mesh=_mesh,
    compiler_params=pltpu.CompilerParams(needs_layout_passes=False),
    scratch_types=[
        pltpu.VMEM((_GC,), jnp.int32),
        pltpu.VMEM((_GC,), jnp.int32),
        pltpu.VMEM((_GC,), jnp.int32),
        pltpu.VMEM((_GC, _D), jnp.float32),
        pltpu.VMEM((_GC, _D), jnp.float32),
        pltpu.VMEM((_GC, _D), jnp.float32),
        pltpu.SemaphoreType.DMA,
        pltpu.SemaphoreType.DMA,
        pltpu.SemaphoreType.DMA,
        pltpu.SemaphoreType.DMA,
        pltpu.SemaphoreType.DMA,
        pltpu.SemaphoreType.DMA,
        pltpu.SemaphoreType.DMA,
        pltpu.SemaphoreType.DMA,
        pltpu.SemaphoreType.DMA,
    ],
)(_gather_body)


def _ffn_body(bexp_ref, xs_ref, w1_ref, w3_ref, w2_ref, y_ref,
              w1b, w3b, w2b):
    h = pl.program_id(0)
    b = pl.program_id(1)
    bprev = jnp.maximum(b - 1, 0)
    fresh = jnp.logical_or(
        b == 0,
        bexp_ref[bprev] != bexp_ref[b])

    @pl.when(fresh)
    def _():
        w1b[...] = w1_ref[0].astype(jnp.bfloat16)
        w3b[...] = w3_ref[0].astype(jnp.bfloat16)
        w2b[...] = w2_ref[0].astype(jnp.bfloat16)

    @pl.when(b < bexp_ref[64])
    def _():
        xb = xs_ref[...].astype(jnp.bfloat16)
        a = lax.dot_general(xb, w1b[...], (((1,), (1,)), ((), ())),
                            preferred_element_type=jnp.float32)
        g = lax.dot_general(xb, w3b[...], (((1,), (1,)), ((), ())),
                            preferred_element_type=jnp.float32)
        hh = (a / (1.0 + jnp.exp(-a))) * g
        y = lax.dot_general(hh.astype(jnp.bfloat16), w2b[...],
                            (((1,), (1,)), ((), ())),
                            preferred_element_type=jnp.float32)
        sl = pl.ds(b * _B, _B)

        @pl.when(h == 0)
        def _():
            y_ref[sl, :] = y

        @pl.when(h > 0)
        def _():
            y_ref[sl, :] = y_ref[sl, :] + y


def _combine_body(y_hbm, ppos_hbm, wf_hbm, out_hbm,
                  idx0, idx1, wv0, wv1, rows0, rows1, o0, o1,
                  gs0, gs1, ws0, ws1):
    wid = lax.axis_index("s") * 2 + lax.axis_index("c")
    idxs = [idx0, idx1]
    wvs = [wv0, wv1]
    rows = [rows0, rows1]
    outs = [o0, o1]
    gsem = [gs0, gs1]
    wsem = [ws0, ws1]
    gcp = [None] * 4
    wcp = [None] * 4

    def flush(c):
        b = c % 2
        gcp[c].wait()
        w_v = wvs[b]
        rows_v = rows[b]
        out_v = outs[b]

        def tok_step(ti, carry):
            w0 = plsc.load_gather(w_v, [jnp.full((16,), 0, jnp.int32) + ti])
            w1 = plsc.load_gather(w_v, [jnp.full((16,), 16, jnp.int32) + ti])
            for cc in range(_D // 16):
                sl = pl.ds(cc * 16, 16)
                out_v[ti, sl] = w0 * rows_v[ti, sl] + w1 * rows_v[ti + 16, sl]
            return carry

        lax.fori_loop(0, 16, tok_step, 0)
        wcp[c] = pltpu.async_copy(
            out_v, out_hbm.at[pl.ds(wid * 64 + c * 16, 16), :], wsem[b])

    for c in range(4):
        b = c % 2
        if c >= 2:
            wcp[c - 2].wait()
        tb = wid * 64 + c * 16
        pltpu.sync_copy(ppos_hbm.at[pl.ds(tb, 16)], idxs[b].at[pl.ds(0, 16)])
        pltpu.sync_copy(ppos_hbm.at[pl.ds(_T + tb, 16)],
                        idxs[b].at[pl.ds(16, 16)])
        pltpu.sync_copy(wf_hbm.at[pl.ds(tb, 16)], wvs[b].at[pl.ds(0, 16)])
        pltpu.sync_copy(wf_hbm.at[pl.ds(_T + tb, 16)],
                        wvs[b].at[pl.ds(16, 16)])
        gcp[c] = pltpu.async_copy(y_hbm.at[idxs[b]], rows[b], gsem[b])
        if c >= 1:
            flush(c - 1)
    flush(3)
    wcp[2].wait()
    wcp[3].wait()


_combine_call = functools.partial(
    pl.kernel,
    out_type=jax.ShapeDtypeStruct((_T, _D), jnp.float32),
    mesh=_mesh,
    compiler_params=pltpu.CompilerParams(needs_layout_passes=False),
    scratch_types=[
        pltpu.VMEM((2 * 16,), jnp.int32),
        pltpu.VMEM((2 * 16,), jnp.int32),
        pltpu.VMEM((2 * 16,), jnp.float32),
        pltpu.VMEM((2 * 16,), jnp.float32),
        pltpu.VMEM((2 * 16, _D), jnp.float32),
        pltpu.VMEM((2 * 16, _D), jnp.float32),
        pltpu.VMEM((16, _D), jnp.float32),
        pltpu.VMEM((16, _D), jnp.float32),
        pltpu.SemaphoreType.DMA,
        pltpu.SemaphoreType.DMA,
        pltpu.SemaphoreType.DMA,
        pltpu.SemaphoreType.DMA,
    ],
)(_combine_body)


def kernel(x, Wg, W1, W2, W3):
    eidx, wf = pl.pallas_call(
        _gate_body,
        out_shape=(jax.ShapeDtypeStruct((_K, _T), jnp.int32),
                   jax.ShapeDtypeStruct((_K, _T), jnp.float32)),
    )(x, Wg)

    ppos, rtok, bexp = _route_call(eidx.reshape(_S))

    xs = _gather_call(x, rtok)

    grid_spec = pltpu.PrefetchScalarGridSpec(
        num_scalar_prefetch=1,
        grid=(_NH, _NB),
        in_specs=[
            pl.BlockSpec((_B, _D), lambda h, b, sref: (b, 0)),
            pl.BlockSpec((1, _HB, _D),
                         lambda h, b, sref: (sref[b], h, 0)),
            pl.BlockSpec((1, _HB, _D),
                         lambda h, b, sref: (sref[b], h, 0)),
            pl.BlockSpec((1, _D, _HB),
                         lambda h, b, sref: (sref[b], 0, h)),
        ],
        out_specs=pl.BlockSpec((_NR, _D), lambda h, b, sref: (0, 0)),
        scratch_shapes=[
            pltpu.VMEM((_HB, _D), jnp.bfloat16),
            pltpu.VMEM((_HB, _D), jnp.bfloat16),
            pltpu.VMEM((_D, _HB), jnp.bfloat16),
        ],
    )
    y = pl.pallas_call(
        _ffn_body,
        grid_spec=grid_spec,
        out_shape=jax.ShapeDtypeStruct((_NR, _D), jnp.float32),
        compiler_params=pltpu.CompilerParams(
            dimension_semantics=("arbitrary", "arbitrary")),
    )(bexp, xs, W1, W3, W2)

    out = _combine_call(y, ppos, wf.reshape(_S))
    return out
